# Initial kernel scaffold; baseline (speedup 1.0000x reference)
import jax
import jax.numpy as jnp
from jax.experimental import pallas as pl


def kernel(hidden_states, routing_weights, selected_experts, gate_up_proj, down_proj):
    raise NotImplementedError("write your pallas kernel here")


# speedup vs baseline: 1.7667x; 1.7667x over previous
import functools

import jax
import jax.numpy as jnp
from jax.experimental import pallas as pl
from jax.experimental.pallas import tpu as pltpu

NUM_EXPERTS = 8
TOP_K = 2
HIDDEN = 1024
INTER = 2816
TOKENS = 2048

PAIRS = TOKENS * TOP_K
BN = 256
NB = INTER // BN
RC = 256


def _moe_body(counts_ref, starts_ref, tok_ref, w_ref,
              hs_ref, wg_ref, wu_ref, wd_ref, out_ref,
              x_s, acc_s):
    e = pl.program_id(0)
    n = pl.program_id(1)
    cnt = counts_ref[e]
    start = starts_ref[e]
    nch = (cnt + RC - 1) // RC

    @pl.when((e == 0) & (n == 0))
    def _init_out():
        out_ref[...] = jnp.zeros_like(out_ref)

    @pl.when(n == 0)
    def _gather():
        def gbody(i, _):
            tok = tok_ref[start + i]
            x_s[pl.ds(i, 1), :] = hs_ref[pl.ds(tok, 1), :]
            return 0
        jax.lax.fori_loop(0, cnt, gbody, 0)

    wg = wg_ref[0]
    wu = wu_ref[0]
    wd = wd_ref[0]

    def chunk(j, add):
        xj = x_s[pl.ds(j * RC, RC), :]
        gate = jnp.dot(xj, wg, preferred_element_type=jnp.float32)
        up = jnp.dot(xj, wu, preferred_element_type=jnp.float32)
        h = gate * jax.nn.sigmoid(gate) * up
        y = jnp.dot(h, wd, preferred_element_type=jnp.float32)
        if add:
            acc_s[pl.ds(j * RC, RC), :] += y
        else:
            acc_s[pl.ds(j * RC, RC), :] = y
        return 0

    @pl.when(n == 0)
    def _first():
        jax.lax.fori_loop(0, nch, lambda j, c: chunk(j, False), 0)

    @pl.when(n != 0)
    def _rest():
        jax.lax.fori_loop(0, nch, lambda j, c: chunk(j, True), 0)

    @pl.when(n == NB - 1)
    def _scatter():
        def sbody(i, _):
            tok = tok_ref[start + i]
            wv = w_ref[start + i]
            out_ref[pl.ds(tok, 1), :] += wv * acc_s[pl.ds(i, 1), :]
            return 0
        jax.lax.fori_loop(0, cnt, sbody, 0)


@jax.jit
def kernel(hidden_states, routing_weights, selected_experts, gate_up_proj, down_proj):
    flat_e = selected_experts.reshape(-1)
    order = jnp.argsort(flat_e)
    sorted_tok = (order // TOP_K).astype(jnp.int32)
    sorted_w = routing_weights.reshape(-1)[order]
    counts = jnp.bincount(flat_e, length=NUM_EXPERTS).astype(jnp.int32)
    starts = (jnp.cumsum(counts) - counts).astype(jnp.int32)

    grid_spec = pltpu.PrefetchScalarGridSpec(
        num_scalar_prefetch=4,
        grid=(NUM_EXPERTS, NB),
        in_specs=[
            pl.BlockSpec((TOKENS, HIDDEN), lambda e, n, *_: (0, 0)),
            pl.BlockSpec((1, HIDDEN, BN), lambda e, n, *_: (e, 0, n)),
            pl.BlockSpec((1, HIDDEN, BN), lambda e, n, *_: (e, 0, n + NB)),
            pl.BlockSpec((1, BN, HIDDEN), lambda e, n, *_: (e, n, 0)),
        ],
        out_specs=pl.BlockSpec((TOKENS, HIDDEN), lambda e, n, *_: (0, 0)),
        scratch_shapes=[
            pltpu.VMEM((PAIRS, HIDDEN), jnp.float32),
            pltpu.VMEM((PAIRS, HIDDEN), jnp.float32),
        ],
    )

    out = pl.pallas_call(
        _moe_body,
        grid_spec=grid_spec,
        out_shape=jax.ShapeDtypeStruct((TOKENS, HIDDEN), jnp.float32),
        compiler_params=pltpu.CompilerParams(
            dimension_semantics=("arbitrary", "arbitrary"),
        ),
    )(counts, starts, sorted_tok, sorted_w,
      hidden_states, gate_up_proj, gate_up_proj, down_proj)
    return out


# speedup vs baseline: 1.7743x; 1.0043x over previous
import functools

import jax
import jax.numpy as jnp
from jax.experimental import pallas as pl
from jax.experimental.pallas import tpu as pltpu

NUM_EXPERTS = 8
TOP_K = 2
HIDDEN = 1024
INTER = 2816
TOKENS = 2048

PAIRS = TOKENS * TOP_K
BN = 256
NB = INTER // BN
RC = 256


def _moe_body(counts_ref, starts_ref, tok_ref, w_ref,
              hs_ref, wg_ref, wu_ref, wd_ref, out_ref,
              x_s, acc_s):
    e = pl.program_id(0)
    n = pl.program_id(1)
    cnt = counts_ref[e]
    start = starts_ref[e]
    nch = (cnt + RC - 1) // RC

    @pl.when((e == 0) & (n == 0))
    def _init_out():
        out_ref[...] = jnp.zeros_like(out_ref)

    @pl.when(n == 0)
    def _gather():
        def gbody(i, _):
            tok = tok_ref[start + i]
            x_s[pl.ds(i, 1), :] = hs_ref[pl.ds(tok, 1), :]
            return 0
        jax.lax.fori_loop(0, cnt, gbody, 0)

    wg = wg_ref[0].astype(jnp.bfloat16)
    wu = wu_ref[0].astype(jnp.bfloat16)
    wd = wd_ref[0].astype(jnp.bfloat16)

    def chunk(j, add):
        xj = x_s[pl.ds(j * RC, RC), :].astype(jnp.bfloat16)
        gate = jnp.dot(xj, wg, preferred_element_type=jnp.float32)
        up = jnp.dot(xj, wu, preferred_element_type=jnp.float32)
        h = (gate * jax.nn.sigmoid(gate) * up).astype(jnp.bfloat16)
        y = jnp.dot(h, wd, preferred_element_type=jnp.float32)
        if add:
            acc_s[pl.ds(j * RC, RC), :] += y
        else:
            acc_s[pl.ds(j * RC, RC), :] = y
        return 0

    @pl.when(n == 0)
    def _first():
        jax.lax.fori_loop(0, nch, lambda j, c: chunk(j, False), 0)

    @pl.when(n != 0)
    def _rest():
        jax.lax.fori_loop(0, nch, lambda j, c: chunk(j, True), 0)

    @pl.when(n == NB - 1)
    def _scatter():
        def sbody(i, _):
            tok = tok_ref[start + i]
            wv = w_ref[start + i]
            out_ref[pl.ds(tok, 1), :] += wv * acc_s[pl.ds(i, 1), :]
            return 0
        jax.lax.fori_loop(0, cnt, sbody, 0)


@jax.jit
def kernel(hidden_states, routing_weights, selected_experts, gate_up_proj, down_proj):
    flat_e = selected_experts.reshape(-1)
    order = jnp.argsort(flat_e)
    sorted_tok = (order // TOP_K).astype(jnp.int32)
    sorted_w = routing_weights.reshape(-1)[order]
    counts = jnp.bincount(flat_e, length=NUM_EXPERTS).astype(jnp.int32)
    starts = (jnp.cumsum(counts) - counts).astype(jnp.int32)

    grid_spec = pltpu.PrefetchScalarGridSpec(
        num_scalar_prefetch=4,
        grid=(NUM_EXPERTS, NB),
        in_specs=[
            pl.BlockSpec((TOKENS, HIDDEN), lambda e, n, *_: (0, 0)),
            pl.BlockSpec((1, HIDDEN, BN), lambda e, n, *_: (e, 0, n)),
            pl.BlockSpec((1, HIDDEN, BN), lambda e, n, *_: (e, 0, n + NB)),
            pl.BlockSpec((1, BN, HIDDEN), lambda e, n, *_: (e, n, 0)),
        ],
        out_specs=pl.BlockSpec((TOKENS, HIDDEN), lambda e, n, *_: (0, 0)),
        scratch_shapes=[
            pltpu.VMEM((PAIRS, HIDDEN), jnp.float32),
            pltpu.VMEM((PAIRS, HIDDEN), jnp.float32),
        ],
    )

    out = pl.pallas_call(
        _moe_body,
        grid_spec=grid_spec,
        out_shape=jax.ShapeDtypeStruct((TOKENS, HIDDEN), jnp.float32),
        compiler_params=pltpu.CompilerParams(
            dimension_semantics=("arbitrary", "arbitrary"),
        ),
    )(counts, starts, sorted_tok, sorted_w,
      hidden_states, gate_up_proj, gate_up_proj, down_proj)
    return out
